# baseline (speedup 1.0000x reference)
import functools

import jax
import jax.numpy as jnp
from jax import lax
from jax.experimental import pallas as pl
from jax.experimental.pallas import tpu as pltpu
from jax.experimental.pallas import tpu_sc as plsc

B = 2
S = 4096
D = 1024
LANES = 16
GROUPS = D // LANES

NC = 2
NS = 16
NW = NC * NS

N_TOTAL = B * S
PER_W = N_TOTAL // NW
CHUNK = 32
NBUF = 3
SLACK = 1
NCHUNK = PER_W // CHUNK


def _emb_kernel(idx_hbm, table_hbm, tt_hbm, out_hbm, idx_v, *scr):
    rows = list(scr[0:NBUF])
    tt_v = scr[NBUF]
    gsem = list(scr[NBUF + 1:2 * NBUF + 1])
    ssem = list(scr[2 * NBUF + 1:3 * NBUF + 1])

    sid = lax.axis_index("s")
    wid = sid * NC + lax.axis_index("c")
    base = wid * PER_W

    pltpu.sync_copy(idx_hbm.at[wid], idx_v)
    pltpu.sync_copy(tt_hbm.at[0], tt_v)

    def start_gather(ci, b):
        return pltpu.async_copy(
            table_hbm.at[idx_v.at[ci]], rows[b], gsem[b])

    def add_tt(b):
        rows_b = rows[b]

        @plsc.parallel_loop(0, GROUPS, step=1, unroll=4)
        def _add(g):
            sl = pl.ds(g * LANES, LANES)
            ttg = tt_v[sl]
            for r in range(CHUNK):
                rows_b[r, sl] = rows_b[r, sl] + ttg

    gathers = [None] * NBUF
    scatters = [None] * NBUF
    for b in range(NBUF):
        gathers[b] = start_gather(b, b)

    for ci in range(NCHUNK):
        b = ci % NBUF
        gathers[b].wait()
        j = ci - SLACK
        if j >= 0 and j + NBUF < NCHUNK:
            bj = j % NBUF
            scatters[bj].wait()
            gathers[bj] = start_gather(j + NBUF, bj)
        add_tt(b)
        scatters[b] = pltpu.async_copy(
            rows[b], out_hbm.at[pl.ds(base + ci * CHUNK, CHUNK)], ssem[b])

    for ci in range(max(0, NCHUNK - NBUF), NCHUNK):
        scatters[ci % NBUF].wait()


@jax.jit
def _emb(flat_ids, word_table, token_type_table):
    run = functools.partial(
        pl.kernel,
        mesh=plsc.VectorSubcoreMesh(core_axis_name="c", subcore_axis_name="s"),
        out_type=jax.ShapeDtypeStruct((N_TOTAL, D), jnp.float32),
        scratch_types=(
            [pltpu.VMEM((NCHUNK, CHUNK), jnp.int32)]
            + [pltpu.VMEM((CHUNK, D), jnp.float32) for _ in range(NBUF)]
            + [pltpu.VMEM((D,), jnp.float32)]
            + [pltpu.SemaphoreType.DMA for _ in range(2 * NBUF)]
        ),
    )(_emb_kernel)
    return run(flat_ids, word_table, token_type_table)


def kernel(input_ids, word_table, token_type_table):
    flat_ids = input_ids.reshape(NW, NCHUNK, CHUNK).astype(jnp.int32)
    out = _emb(flat_ids, word_table, token_type_table)
    return out.reshape(B, S, D)


# speedup vs baseline: 1.0276x; 1.0276x over previous
import functools

import jax
import jax.numpy as jnp
from jax import lax
from jax.experimental import pallas as pl
from jax.experimental.pallas import tpu as pltpu
from jax.experimental.pallas import tpu_sc as plsc

B = 2
S = 4096
D = 1024
LANES = 16
GROUPS = D // LANES

NC = 2
NS = 16
NW = NC * NS

N_TOTAL = B * S
PER_W = N_TOTAL // NW
CHUNK = 16
NBUF = 7
SLACK = 2
NCHUNK = PER_W // CHUNK


def _emb_kernel(idx_hbm, table_hbm, tt_hbm, out_hbm, idx_v, *scr):
    rows = list(scr[0:NBUF])
    tt_v = scr[NBUF]
    gsem = list(scr[NBUF + 1:2 * NBUF + 1])
    ssem = list(scr[2 * NBUF + 1:3 * NBUF + 1])

    sid = lax.axis_index("s")
    wid = sid * NC + lax.axis_index("c")
    base = wid * PER_W

    pltpu.sync_copy(idx_hbm.at[wid], idx_v)
    pltpu.sync_copy(tt_hbm.at[0], tt_v)

    def start_gather(ci, b):
        return pltpu.async_copy(
            table_hbm.at[idx_v.at[ci]], rows[b], gsem[b])

    def add_tt(b, r_lo, r_hi):
        rows_b = rows[b]

        @plsc.parallel_loop(0, GROUPS, step=1, unroll=8)
        def _add(g):
            sl = pl.ds(g * LANES, LANES)
            ttg = tt_v[sl]
            for r in range(r_lo, r_hi):
                rows_b[r, sl] = rows_b[r, sl] + ttg

    gathers = [None] * NBUF
    scatters = [None] * NBUF
    for b in range(NBUF):
        gathers[b] = start_gather(b, b)

    for ci in range(NCHUNK):
        b = ci % NBUF
        gathers[b].wait()
        j = ci - SLACK
        if j >= 0 and j + NBUF < NCHUNK:
            bj = j % NBUF
            for d in scatters[bj]:
                d.wait()
            gathers[bj] = start_gather(j + NBUF, bj)
        half = CHUNK // 2
        add_tt(b, 0, half)
        s_lo = pltpu.async_copy(
            rows[b].at[pl.ds(0, half)],
            out_hbm.at[pl.ds(base + ci * CHUNK, half)], ssem[b])
        add_tt(b, half, CHUNK)
        s_hi = pltpu.async_copy(
            rows[b].at[pl.ds(half, half)],
            out_hbm.at[pl.ds(base + ci * CHUNK + half, half)], ssem[b])
        scatters[b] = [s_lo, s_hi]

    for ci in range(max(0, NCHUNK - NBUF), NCHUNK):
        for d in scatters[ci % NBUF]:
            d.wait()


@jax.jit
def _emb(flat_ids, word_table, token_type_table):
    run = functools.partial(
        pl.kernel,
        mesh=plsc.VectorSubcoreMesh(core_axis_name="c", subcore_axis_name="s"),
        out_type=jax.ShapeDtypeStruct((N_TOTAL, D), jnp.float32),
        scratch_types=(
            [pltpu.VMEM((NCHUNK, CHUNK), jnp.int32)]
            + [pltpu.VMEM((CHUNK, D), jnp.float32) for _ in range(NBUF)]
            + [pltpu.VMEM((D,), jnp.float32)]
            + [pltpu.SemaphoreType.DMA for _ in range(2 * NBUF)]
        ),
    )(_emb_kernel)
    return run(flat_ids, word_table, token_type_table)


def kernel(input_ids, word_table, token_type_table):
    flat_ids = input_ids.reshape(NW, NCHUNK, CHUNK).astype(jnp.int32)
    out = _emb(flat_ids, word_table, token_type_table)
    return out.reshape(B, S, D)


# speedup vs baseline: 1.1290x; 1.0987x over previous
import functools

import jax
import jax.numpy as jnp
from jax import lax
from jax.experimental import pallas as pl
from jax.experimental.pallas import tpu as pltpu
from jax.experimental.pallas import tpu_sc as plsc

B = 2
S = 4096
D = 1024
LANES = 16
GROUPS = D // LANES

NC = 2
NS = 16
NW = NC * NS

N_TOTAL = B * S
PER_W = N_TOTAL // NW
CHUNK = 16
NBUF = 7
SLACK = 2
NCHUNK = PER_W // CHUNK


def _emb_kernel(idx_hbm, table_hbm, tt_hbm, out_hbm, idx_v, *scr):
    rows = list(scr[0:NBUF])
    tt_v = scr[NBUF]
    gsem = list(scr[NBUF + 1:2 * NBUF + 1])
    ssem = list(scr[2 * NBUF + 1:3 * NBUF + 1])

    sid = lax.axis_index("s")
    wid = sid * NC + lax.axis_index("c")
    base = wid * PER_W

    pltpu.sync_copy(idx_hbm.at[wid], idx_v)
    pltpu.sync_copy(tt_hbm.at[0], tt_v)

    def start_gather(ci, b):
        return pltpu.async_copy(
            table_hbm.at[idx_v.at[ci]], rows[b], gsem[b])

    def add_tt(b):
        rows_b = rows[b]

        @plsc.parallel_loop(0, GROUPS, step=1, unroll=8)
        def _add(g):
            sl = pl.ds(g * LANES, LANES)
            ttg = tt_v[sl]
            for r in range(CHUNK):
                rows_b[r, sl] = rows_b[r, sl] + ttg

    gathers = [None] * NBUF
    scatters = [None] * NBUF
    for b in range(NBUF):
        gathers[b] = start_gather(b, b)

    for ci in range(NCHUNK):
        b = ci % NBUF
        gathers[b].wait()
        j = ci - SLACK
        if j >= 0 and j + NBUF < NCHUNK:
            bj = j % NBUF
            scatters[bj].wait()
            gathers[bj] = start_gather(j + NBUF, bj)
        add_tt(b)
        scatters[b] = pltpu.async_copy(
            rows[b], out_hbm.at[pl.ds(base + ci * CHUNK, CHUNK)], ssem[b])

    for ci in range(max(0, NCHUNK - NBUF), NCHUNK):
        scatters[ci % NBUF].wait()


@jax.jit
def _emb(flat_ids, word_table, token_type_table):
    run = functools.partial(
        pl.kernel,
        mesh=plsc.VectorSubcoreMesh(core_axis_name="c", subcore_axis_name="s"),
        out_type=jax.ShapeDtypeStruct((N_TOTAL, D), jnp.float32),
        scratch_types=(
            [pltpu.VMEM((NCHUNK, CHUNK), jnp.int32)]
            + [pltpu.VMEM((CHUNK, D), jnp.float32) for _ in range(NBUF)]
            + [pltpu.VMEM((D,), jnp.float32)]
            + [pltpu.SemaphoreType.DMA for _ in range(2 * NBUF)]
        ),
    )(_emb_kernel)
    return run(flat_ids, word_table, token_type_table)


def kernel(input_ids, word_table, token_type_table):
    flat_ids = input_ids.reshape(NW, NCHUNK, CHUNK).astype(jnp.int32)
    out = _emb(flat_ids, word_table, token_type_table)
    return out.reshape(B, S, D)
